# baseline (speedup 1.0000x reference)
import functools

import jax
import jax.numpy as jnp
from jax import lax
from jax.experimental import pallas as pl
from jax.experimental.pallas import tpu as pltpu
from jax.experimental.pallas import tpu_sc as plsc

N = 50000
E = 800000
C = 64

NC = 2
NS = 16
HALF = N // NC
ACC_ROWS = HALF + NS

EPS = E // NS
CHUNK = 128
CPB = 3
IDXBLK = 384
NBLK = EPS // IDXBLK
TAIL = EPS - NBLK * IDXBLK
CCAP = 528

INIT_SZ = 1568
INIT_LAST = HALF - (NS - 1) * INIT_SZ

ROWBLK = 5000


def _sc_body(x_hbm, src_hbm, tgt_hbm, agg_hbm,
             acc, rowbufs, sblk, tblk, csrc, ctgt, s2d, t2d,
             isem, gsems, ssems):
    c = lax.axis_index("c")
    s = lax.axis_index("s")
    base_node = c * HALF
    dummy = HALF + s

    @pl.when(s < NS - 1)
    def _():
        pltpu.sync_copy(x_hbm.at[pl.ds(base_node + s * INIT_SZ, INIT_SZ)],
                        acc.at[pl.ds(s * INIT_SZ, INIT_SZ)])

    @pl.when(s == NS - 1)
    def _():
        pltpu.sync_copy(x_hbm.at[pl.ds(base_node + (NS - 1) * INIT_SZ, INIT_LAST)],
                        acc.at[pl.ds((NS - 1) * INIT_SZ, INIT_LAST)])

    plsc.subcore_barrier()

    e0 = s * EPS

    def load_idx(p, blk, n):
        eb = e0 + blk * IDXBLK
        pltpu.async_copy(src_hbm.at[pl.ds(eb, n)], sblk.at[p].at[pl.ds(0, n)], isem)
        pltpu.async_copy(tgt_hbm.at[pl.ds(eb, n)], tblk.at[p].at[pl.ds(0, n)], isem)

    def drain_idx(p, n):
        pltpu.make_async_copy(src_hbm.at[pl.ds(e0, n)],
                              sblk.at[p].at[pl.ds(0, n)], isem).wait()
        pltpu.make_async_copy(tgt_hbm.at[pl.ds(e0, n)],
                              tblk.at[p].at[pl.ds(0, n)], isem).wait()

    iota16 = lax.iota(jnp.int32, 16)

    def compact(p, mvec_in, ngroups):
        mvec = mvec_in
        for i in range(ngroups):
            sv = sblk[p, pl.ds(i * 16, 16)]
            t = tblk[p, pl.ds(i * 16, 16)]
            tl = t - base_node
            ok = (tl >= 0) & (tl < HALF)
            key = jnp.where(ok, iota16, iota16 + 16)
            _, sv_c = plsc.sort_key_val(key, sv)
            _, tl_c = plsc.sort_key_val(key, tl)
            pos = mvec + iota16
            plsc.store_scatter(csrc, (pos,), sv_c)
            plsc.store_scatter(ctgt, (pos,), tl_c)
            mvec = mvec + plsc.all_reduce_population_count(ok)
        return mvec

    def fire_gather(k):
        return pltpu.async_copy(
            x_hbm.at[s2d.at[k]], rowbufs[k], gsems[k])

    def wait_gather(k):
        pltpu.make_async_copy(
            x_hbm.at[s2d.at[k]], rowbufs[k], gsems[k]).wait()

    def fire_scatter(k):
        pltpu.async_copy(rowbufs[k], acc.at[t2d.at[k]], ssems[k], add=True)

    def drain_scatter(k):
        pltpu.make_async_copy(rowbufs[k], acc.at[t2d.at[k]], ssems[k]).wait()

    def fire_block(m_tot, pend):
        nfire = m_tot // CHUNK
        for k in range(CPB):
            @pl.when(k < nfire)
            def _():
                @pl.when(((pend >> k) & 1) == 1)
                def _():
                    drain_scatter(k)
                for ii in range(CHUNK // 16):
                    s2d[k, pl.ds(ii * 16, 16)] = csrc[pl.ds(k * CHUNK + ii * 16, 16)]
                    t2d[k, pl.ds(ii * 16, 16)] = ctgt[pl.ds(k * CHUNK + ii * 16, 16)]
                fire_gather(k)

        mrem = m_tot - nfire * CHUNK

        @pl.when(nfire > 0)
        def _():
            for i in range(CHUNK // 16):
                @pl.when(i * 16 < mrem)
                def _():
                    src_pos = nfire * CHUNK + i * 16 + iota16
                    csrc[pl.ds(i * 16, 16)] = plsc.load_gather(csrc, (src_pos,))
                    ctgt[pl.ds(i * 16, 16)] = plsc.load_gather(ctgt, (src_pos,))

        pend_out = pend & ~((jnp.int32(1) << nfire) - 1)
        return (nfire.astype(jnp.int32), mrem.astype(jnp.int32),
                pend_out.astype(jnp.int32))

    def scatter_block(nprev, pend):
        for k in range(CPB):
            @pl.when(k < nprev)
            def _():
                wait_gather(k)
                fire_scatter(k)
        return (pend | ((jnp.int32(1) << nprev) - 1)).astype(jnp.int32)

    pltpu.sync_copy(src_hbm.at[pl.ds(e0, IDXBLK)], sblk.at[0])
    pltpu.sync_copy(tgt_hbm.at[pl.ds(e0, IDXBLK)], tblk.at[0])
    load_idx(1, 1, IDXBLK)
    mvec = compact(0, jnp.zeros((16,), jnp.int32), IDXBLK // 16)
    nfire, _, pend = fire_block(jnp.max(mvec), jnp.int32(0))
    mvec = mvec - nfire * CHUNK

    def body(g, carry):
        mvec, nprev, pend = carry
        p = g % 2
        q = 1 - p
        drain_idx(p, IDXBLK)

        @pl.when(g + 1 < NBLK)
        def _():
            load_idx(q, g + 1, IDXBLK)

        pend = scatter_block(nprev, pend)
        mvec = compact(p, mvec, IDXBLK // 16)
        nfire, _, pend = fire_block(jnp.max(mvec), pend)
        return mvec - nfire * CHUNK, nfire, pend

    mvec, nprev, pend = lax.fori_loop(1, NBLK, body, (mvec, nfire, pend))

    pend = scatter_block(nprev, pend)

    et = e0 + NBLK * IDXBLK
    pltpu.sync_copy(src_hbm.at[pl.ds(et, TAIL)], sblk.at[0].at[pl.ds(0, TAIL)])
    pltpu.sync_copy(tgt_hbm.at[pl.ds(et, TAIL)], tblk.at[0].at[pl.ds(0, TAIL)])
    mvec = compact(0, mvec, TAIL // 16)
    m_tot = jnp.max(mvec)
    nfire2 = (m_tot + CHUNK - 1) // CHUNK
    for i in range((2 * CHUNK) // 16):
        lane = lax.iota(jnp.int32, 16) + (i * 16)
        inside = lane < m_tot
        csrc[pl.ds(i * 16, 16)] = jnp.where(inside, csrc[pl.ds(i * 16, 16)], 0)
        ctgt[pl.ds(i * 16, 16)] = jnp.where(inside, ctgt[pl.ds(i * 16, 16)], dummy)
    for k in range(2):
        @pl.when(((pend >> k) & 1) == 1)
        def _():
            drain_scatter(k)
    pend = pend & ~3
    for k in range(2):
        @pl.when(k < nfire2)
        def _():
            for ii in range(CHUNK // 16):
                s2d[k, pl.ds(ii * 16, 16)] = csrc[pl.ds(k * CHUNK + ii * 16, 16)]
                t2d[k, pl.ds(ii * 16, 16)] = ctgt[pl.ds(k * CHUNK + ii * 16, 16)]
            fire_gather(k)
    for k in range(2):
        @pl.when(k < nfire2)
        def _():
            wait_gather(k)
            fire_scatter(k)
    pend = pend | ((jnp.int32(1) << nfire2) - 1)

    for k in range(CPB):
        @pl.when(((pend >> k) & 1) == 1)
        def _():
            drain_scatter(k)

    plsc.subcore_barrier()

    @pl.when(s < NS - 1)
    def _():
        pltpu.sync_copy(acc.at[pl.ds(s * INIT_SZ, INIT_SZ)],
                        agg_hbm.at[pl.ds(base_node + s * INIT_SZ, INIT_SZ)])

    @pl.when(s == NS - 1)
    def _():
        pltpu.sync_copy(acc.at[pl.ds((NS - 1) * INIT_SZ, INIT_LAST)],
                        agg_hbm.at[pl.ds(base_node + (NS - 1) * INIT_SZ, INIT_LAST)])


_sc_aggregate = functools.partial(
    pl.kernel,
    out_type=jax.ShapeDtypeStruct((N, C), jnp.float32),
    mesh=plsc.VectorSubcoreMesh(core_axis_name="c", subcore_axis_name="s"),
    compiler_params=pltpu.CompilerParams(use_tc_tiling_on_sc=False,
                                         needs_layout_passes=False),
    scratch_types=[
        pltpu.VMEM_SHARED((ACC_ROWS, C), jnp.float32),
        [pltpu.VMEM((CHUNK, C), jnp.float32)] * CPB,
        pltpu.VMEM((2, IDXBLK), jnp.int32),
        pltpu.VMEM((2, IDXBLK), jnp.int32),
        pltpu.VMEM((CCAP,), jnp.int32),
        pltpu.VMEM((CCAP,), jnp.int32),
        pltpu.VMEM((CPB, CHUNK), jnp.int32),
        pltpu.VMEM((CPB, CHUNK), jnp.int32),
        pltpu.SemaphoreType.DMA,
        [pltpu.SemaphoreType.DMA] * CPB,
        [pltpu.SemaphoreType.DMA] * CPB,
    ],
)(_sc_body)


def _tc_body(norm_ref, agg_ref, w_ref, out_ref):
    h = norm_ref[...] * agg_ref[...]
    out_ref[...] = jnp.dot(h, w_ref[...], preferred_element_type=jnp.float32)


def _tc_matmul(norm, agg, W):
    return pl.pallas_call(
        _tc_body,
        grid=(N // ROWBLK,),
        in_specs=[
            pl.BlockSpec((ROWBLK, 1), lambda i: (i, 0)),
            pl.BlockSpec((ROWBLK, C), lambda i: (i, 0)),
            pl.BlockSpec((C, C), lambda i: (0, 0)),
        ],
        out_specs=pl.BlockSpec((ROWBLK, C), lambda i: (i, 0)),
        out_shape=jax.ShapeDtypeStruct((N, C), jnp.float32),
    )(norm, agg, W)


def kernel(x, sources, targets, norm, W):
    src = sources.astype(jnp.int32)
    tgt = targets.astype(jnp.int32)
    agg = _sc_aggregate(x, src, tgt)
    return _tc_matmul(norm, agg, W)


# speedup vs baseline: 1.0232x; 1.0232x over previous
import functools

import jax
import jax.numpy as jnp
from jax import lax
from jax.experimental import pallas as pl
from jax.experimental.pallas import tpu as pltpu
from jax.experimental.pallas import tpu_sc as plsc

N = 50000
E = 800000
C = 64

NC = 2
NS = 16
HALF = N // NC
ACC_ROWS = HALF + NS

EPS = E // NS
CHUNK = 96
CPB = 4
IDXBLK = CPB * CHUNK
NBLK = EPS // IDXBLK
TAIL = EPS - NBLK * IDXBLK
CCAP = 496
DUMP = 480

INIT_SZ = 1568
INIT_LAST = HALF - (NS - 1) * INIT_SZ

ROWBLK = 5000


def _sc_body(x_hbm, src_hbm, tgt_hbm, agg_hbm,
             acc, rowbufs, sblk, tblk, cpk, s2d, t2d,
             isem, gsems, ssems):
    c = lax.axis_index("c")
    s = lax.axis_index("s")
    base_node = c * HALF
    dummy = HALF + s

    @pl.when(s < NS - 1)
    def _():
        pltpu.sync_copy(x_hbm.at[pl.ds(base_node + s * INIT_SZ, INIT_SZ)],
                        acc.at[pl.ds(s * INIT_SZ, INIT_SZ)])

    @pl.when(s == NS - 1)
    def _():
        pltpu.sync_copy(x_hbm.at[pl.ds(base_node + (NS - 1) * INIT_SZ, INIT_LAST)],
                        acc.at[pl.ds((NS - 1) * INIT_SZ, INIT_LAST)])

    plsc.subcore_barrier()

    e0 = s * EPS

    def load_idx(p, blk, n):
        eb = e0 + blk * IDXBLK
        pltpu.async_copy(src_hbm.at[pl.ds(eb, n)], sblk.at[p].at[pl.ds(0, n)], isem)
        pltpu.async_copy(tgt_hbm.at[pl.ds(eb, n)], tblk.at[p].at[pl.ds(0, n)], isem)

    def drain_idx(p, n):
        pltpu.make_async_copy(src_hbm.at[pl.ds(e0, n)],
                              sblk.at[p].at[pl.ds(0, n)], isem).wait()
        pltpu.make_async_copy(tgt_hbm.at[pl.ds(e0, n)],
                              tblk.at[p].at[pl.ds(0, n)], isem).wait()

    iota16 = lax.iota(jnp.int32, 16)

    def compact(p, mvec_in, ngroups):
        mvec = mvec_in
        for i in range(ngroups):
            sv = sblk[p, pl.ds(i * 16, 16)]
            t = tblk[p, pl.ds(i * 16, 16)]
            tl = t - base_node
            ok = (tl >= 0) & (tl < HALF)
            key = jnp.where(ok, iota16, iota16 + 16)
            _, pk_c = plsc.sort_key_val(key, (sv << 15) | (tl & 32767))
            plsc.store_scatter(cpk, (mvec + iota16,), pk_c)
            mvec = mvec + plsc.all_reduce_population_count(ok)
        return mvec

    def fire_gather(k):
        return pltpu.async_copy(
            x_hbm.at[s2d.at[k]], rowbufs[k], gsems[k])

    def wait_gather(k):
        pltpu.make_async_copy(
            x_hbm.at[s2d.at[k]], rowbufs[k], gsems[k]).wait()

    def fire_scatter(k):
        pltpu.async_copy(rowbufs[k], acc.at[t2d.at[k]], ssems[k], add=True)

    def drain_scatter(k):
        pltpu.make_async_copy(rowbufs[k], acc.at[t2d.at[k]], ssems[k]).wait()

    def fire_block(m_tot, pend):
        nfire = m_tot // CHUNK
        for k in range(CPB):
            @pl.when(k < nfire)
            def _():
                @pl.when(((pend >> k) & 1) == 1)
                def _():
                    drain_scatter(k)
                for ii in range(CHUNK // 16):
                    pk = cpk[pl.ds(k * CHUNK + ii * 16, 16)]
                    s2d[k, pl.ds(ii * 16, 16)] = pk >> 15
                    t2d[k, pl.ds(ii * 16, 16)] = pk & 32767
                fire_gather(k)

        mrem = m_tot - nfire * CHUNK

        @pl.when(nfire > 0)
        def _():
            for i in range(CHUNK // 16):
                @pl.when(i * 16 < mrem)
                def _():
                    src_pos = nfire * CHUNK + i * 16 + iota16
                    cpk[pl.ds(i * 16, 16)] = plsc.load_gather(cpk, (src_pos,))

        pend_out = pend & ~((jnp.int32(1) << nfire) - 1)
        return (nfire.astype(jnp.int32), mrem.astype(jnp.int32),
                pend_out.astype(jnp.int32))

    def scatter_block(nprev, pend):
        for k in range(CPB):
            @pl.when(k < nprev)
            def _():
                wait_gather(k)
                fire_scatter(k)
        return (pend | ((jnp.int32(1) << nprev) - 1)).astype(jnp.int32)

    pltpu.sync_copy(src_hbm.at[pl.ds(e0, IDXBLK)], sblk.at[0])
    pltpu.sync_copy(tgt_hbm.at[pl.ds(e0, IDXBLK)], tblk.at[0])
    load_idx(1, 1, IDXBLK)
    mvec = compact(0, jnp.zeros((16,), jnp.int32), IDXBLK // 16)
    nfire, _, pend = fire_block(jnp.max(mvec), jnp.int32(0))
    mvec = mvec - nfire * CHUNK

    def body(g, carry):
        mvec, nprev, pend = carry
        p = g % 2
        q = 1 - p
        drain_idx(p, IDXBLK)

        @pl.when(g + 1 < NBLK)
        def _():
            load_idx(q, g + 1, IDXBLK)

        pend = scatter_block(nprev, pend)
        mvec = compact(p, mvec, IDXBLK // 16)
        nfire, _, pend = fire_block(jnp.max(mvec), pend)
        return mvec - nfire * CHUNK, nfire, pend

    mvec, nprev, pend = lax.fori_loop(1, NBLK, body, (mvec, nfire, pend))

    pend = scatter_block(nprev, pend)

    et = e0 + NBLK * IDXBLK
    pltpu.sync_copy(src_hbm.at[pl.ds(et, TAIL)], sblk.at[0].at[pl.ds(0, TAIL)])
    pltpu.sync_copy(tgt_hbm.at[pl.ds(et, TAIL)], tblk.at[0].at[pl.ds(0, TAIL)])
    mvec = compact(0, mvec, TAIL // 16)
    m_tot = jnp.max(mvec)
    nfire2 = (m_tot + CHUNK - 1) // CHUNK
    for i in range((2 * CHUNK) // 16):
        lane = lax.iota(jnp.int32, 16) + (i * 16)
        inside = lane < m_tot
        cpk[pl.ds(i * 16, 16)] = jnp.where(inside, cpk[pl.ds(i * 16, 16)], dummy)
    for k in range(2):
        @pl.when(((pend >> k) & 1) == 1)
        def _():
            drain_scatter(k)
    pend = pend & ~3
    for k in range(2):
        @pl.when(k < nfire2)
        def _():
            for ii in range(CHUNK // 16):
                pk = cpk[pl.ds(k * CHUNK + ii * 16, 16)]
                s2d[k, pl.ds(ii * 16, 16)] = pk >> 15
                t2d[k, pl.ds(ii * 16, 16)] = pk & 32767
            fire_gather(k)
    for k in range(2):
        @pl.when(k < nfire2)
        def _():
            wait_gather(k)
            fire_scatter(k)
    pend = pend | ((jnp.int32(1) << nfire2) - 1)

    for k in range(CPB):
        @pl.when(((pend >> k) & 1) == 1)
        def _():
            drain_scatter(k)

    plsc.subcore_barrier()

    @pl.when(s < NS - 1)
    def _():
        pltpu.sync_copy(acc.at[pl.ds(s * INIT_SZ, INIT_SZ)],
                        agg_hbm.at[pl.ds(base_node + s * INIT_SZ, INIT_SZ)])

    @pl.when(s == NS - 1)
    def _():
        pltpu.sync_copy(acc.at[pl.ds((NS - 1) * INIT_SZ, INIT_LAST)],
                        agg_hbm.at[pl.ds(base_node + (NS - 1) * INIT_SZ, INIT_LAST)])


_sc_aggregate = functools.partial(
    pl.kernel,
    out_type=jax.ShapeDtypeStruct((N, C), jnp.float32),
    mesh=plsc.VectorSubcoreMesh(core_axis_name="c", subcore_axis_name="s"),
    compiler_params=pltpu.CompilerParams(use_tc_tiling_on_sc=False,
                                         needs_layout_passes=False),
    scratch_types=[
        pltpu.VMEM_SHARED((ACC_ROWS, C), jnp.float32),
        [pltpu.VMEM((CHUNK, C), jnp.float32)] * CPB,
        pltpu.VMEM((2, IDXBLK), jnp.int32),
        pltpu.VMEM((2, IDXBLK), jnp.int32),
        pltpu.VMEM((CCAP,), jnp.int32),
        pltpu.VMEM((CPB, CHUNK), jnp.int32),
        pltpu.VMEM((CPB, CHUNK), jnp.int32),
        pltpu.SemaphoreType.DMA,
        [pltpu.SemaphoreType.DMA] * CPB,
        [pltpu.SemaphoreType.DMA] * CPB,
    ],
)(_sc_body)


def _tc_body(norm_ref, agg_ref, w_ref, out_ref):
    h = norm_ref[...] * agg_ref[...]
    out_ref[...] = jnp.dot(h, w_ref[...], preferred_element_type=jnp.float32)


def _tc_matmul(norm, agg, W):
    return pl.pallas_call(
        _tc_body,
        grid=(N // ROWBLK,),
        in_specs=[
            pl.BlockSpec((ROWBLK, 1), lambda i: (i, 0)),
            pl.BlockSpec((ROWBLK, C), lambda i: (i, 0)),
            pl.BlockSpec((C, C), lambda i: (0, 0)),
        ],
        out_specs=pl.BlockSpec((ROWBLK, C), lambda i: (i, 0)),
        out_shape=jax.ShapeDtypeStruct((N, C), jnp.float32),
    )(norm, agg, W)


def kernel(x, sources, targets, norm, W):
    src = sources.astype(jnp.int32)
    tgt = targets.astype(jnp.int32)
    agg = _sc_aggregate(x, src, tgt)
    return _tc_matmul(norm, agg, W)


# speedup vs baseline: 1.0314x; 1.0080x over previous
import functools

import jax
import jax.numpy as jnp
from jax import lax
from jax.experimental import pallas as pl
from jax.experimental.pallas import tpu as pltpu
from jax.experimental.pallas import tpu_sc as plsc

N = 50000
E = 800000
C = 64

NC = 2
NS = 16
HALF = N // NC
ACC_ROWS = HALF + NS

EPS = E // NS
CHUNK = 96
CPB = 4
IDXBLK = CPB * CHUNK
NBLK = EPS // IDXBLK
TAIL = EPS - NBLK * IDXBLK
CCAP = 496
DUMP = 480

INIT_SZ = 1568
INIT_LAST = HALF - (NS - 1) * INIT_SZ

ROWBLK = 5000


def _sc_body(x_hbm, src_hbm, tgt_hbm, agg_hbm,
             acc, rowbufs, sblk, tblk, cpk, s2d, t2d,
             isem, gsems, ssems):
    c = lax.axis_index("c")
    s = lax.axis_index("s")
    base_node = c * HALF
    dummy = HALF + s

    @pl.when(s < NS - 1)
    def _():
        pltpu.sync_copy(x_hbm.at[pl.ds(base_node + s * INIT_SZ, INIT_SZ)],
                        acc.at[pl.ds(s * INIT_SZ, INIT_SZ)])

    @pl.when(s == NS - 1)
    def _():
        pltpu.sync_copy(x_hbm.at[pl.ds(base_node + (NS - 1) * INIT_SZ, INIT_LAST)],
                        acc.at[pl.ds((NS - 1) * INIT_SZ, INIT_LAST)])

    plsc.subcore_barrier()

    e0 = s * EPS

    def load_idx(p, blk, n):
        eb = e0 + blk * IDXBLK
        pltpu.async_copy(src_hbm.at[pl.ds(eb, n)], sblk.at[p].at[pl.ds(0, n)], isem)
        pltpu.async_copy(tgt_hbm.at[pl.ds(eb, n)], tblk.at[p].at[pl.ds(0, n)], isem)

    def drain_idx(p, n):
        pltpu.make_async_copy(src_hbm.at[pl.ds(e0, n)],
                              sblk.at[p].at[pl.ds(0, n)], isem).wait()
        pltpu.make_async_copy(tgt_hbm.at[pl.ds(e0, n)],
                              tblk.at[p].at[pl.ds(0, n)], isem).wait()

    iota16 = lax.iota(jnp.int32, 16)

    def compact(p, mvec_in, ngroups, g0=0):
        mvec = mvec_in
        for i in range(g0, g0 + ngroups):
            sv = sblk[p, pl.ds(i * 16, 16)]
            t = tblk[p, pl.ds(i * 16, 16)]
            tl = t - base_node
            ok = (tl >= 0) & (tl < HALF)
            key = jnp.where(ok, iota16, iota16 + 16)
            _, pk_c = plsc.sort_key_val(key, (sv << 15) | (tl & 32767))
            plsc.store_scatter(cpk, (mvec + iota16,), pk_c)
            mvec = mvec + plsc.all_reduce_population_count(ok)
        return mvec

    def fire_gather(k):
        return pltpu.async_copy(
            x_hbm.at[s2d.at[k]], rowbufs[k], gsems[k])

    def wait_gather(k):
        pltpu.make_async_copy(
            x_hbm.at[s2d.at[k]], rowbufs[k], gsems[k]).wait()

    def fire_scatter(k):
        pltpu.async_copy(rowbufs[k], acc.at[t2d.at[k]], ssems[k], add=True)

    def drain_scatter(k):
        pltpu.make_async_copy(rowbufs[k], acc.at[t2d.at[k]], ssems[k]).wait()

    def fire_block(m_tot, pend):
        nfire = m_tot // CHUNK
        for k in range(CPB):
            @pl.when(k < nfire)
            def _():
                @pl.when(((pend >> k) & 1) == 1)
                def _():
                    drain_scatter(k)
                for ii in range(CHUNK // 16):
                    pk = cpk[pl.ds(k * CHUNK + ii * 16, 16)]
                    s2d[k, pl.ds(ii * 16, 16)] = pk >> 15
                    t2d[k, pl.ds(ii * 16, 16)] = pk & 32767
                fire_gather(k)

        mrem = m_tot - nfire * CHUNK

        @pl.when(nfire > 0)
        def _():
            for i in range(CHUNK // 16):
                @pl.when(i * 16 < mrem)
                def _():
                    src_pos = nfire * CHUNK + i * 16 + iota16
                    cpk[pl.ds(i * 16, 16)] = plsc.load_gather(cpk, (src_pos,))

        pend_out = pend & ~((jnp.int32(1) << nfire) - 1)
        return (nfire.astype(jnp.int32), mrem.astype(jnp.int32),
                pend_out.astype(jnp.int32))

    def scatter_block(nprev, pend):
        for k in range(CPB):
            @pl.when(k < nprev)
            def _():
                wait_gather(k)
                fire_scatter(k)
        return (pend | ((jnp.int32(1) << nprev) - 1)).astype(jnp.int32)

    pltpu.sync_copy(src_hbm.at[pl.ds(e0, IDXBLK)], sblk.at[0])
    pltpu.sync_copy(tgt_hbm.at[pl.ds(e0, IDXBLK)], tblk.at[0])
    load_idx(1, 1, IDXBLK)
    mvec = compact(0, jnp.zeros((16,), jnp.int32), IDXBLK // 16)
    nfire, _, pend = fire_block(jnp.max(mvec), jnp.int32(0))
    mvec = mvec - nfire * CHUNK

    def body(g, carry):
        mvec, nprev, pend = carry
        p = g % 2
        q = 1 - p
        drain_idx(p, IDXBLK)

        @pl.when(g + 1 < NBLK)
        def _():
            load_idx(q, g + 1, IDXBLK)

        mvec = compact(p, mvec, IDXBLK // 32)
        pend = scatter_block(nprev, pend)
        mvec = compact(p, mvec, IDXBLK // 16 - IDXBLK // 32, g0=IDXBLK // 32)
        nfire, _, pend = fire_block(jnp.max(mvec), pend)
        return mvec - nfire * CHUNK, nfire, pend

    mvec, nprev, pend = lax.fori_loop(1, NBLK, body, (mvec, nfire, pend))

    pend = scatter_block(nprev, pend)

    et = e0 + NBLK * IDXBLK
    pltpu.sync_copy(src_hbm.at[pl.ds(et, TAIL)], sblk.at[0].at[pl.ds(0, TAIL)])
    pltpu.sync_copy(tgt_hbm.at[pl.ds(et, TAIL)], tblk.at[0].at[pl.ds(0, TAIL)])
    mvec = compact(0, mvec, TAIL // 16)
    m_tot = jnp.max(mvec)
    nfire2 = (m_tot + CHUNK - 1) // CHUNK
    for i in range((2 * CHUNK) // 16):
        lane = lax.iota(jnp.int32, 16) + (i * 16)
        inside = lane < m_tot
        cpk[pl.ds(i * 16, 16)] = jnp.where(inside, cpk[pl.ds(i * 16, 16)], dummy)
    for k in range(2):
        @pl.when(((pend >> k) & 1) == 1)
        def _():
            drain_scatter(k)
    pend = pend & ~3
    for k in range(2):
        @pl.when(k < nfire2)
        def _():
            for ii in range(CHUNK // 16):
                pk = cpk[pl.ds(k * CHUNK + ii * 16, 16)]
                s2d[k, pl.ds(ii * 16, 16)] = pk >> 15
                t2d[k, pl.ds(ii * 16, 16)] = pk & 32767
            fire_gather(k)
    for k in range(2):
        @pl.when(k < nfire2)
        def _():
            wait_gather(k)
            fire_scatter(k)
    pend = pend | ((jnp.int32(1) << nfire2) - 1)

    for k in range(CPB):
        @pl.when(((pend >> k) & 1) == 1)
        def _():
            drain_scatter(k)

    plsc.subcore_barrier()

    @pl.when(s < NS - 1)
    def _():
        pltpu.sync_copy(acc.at[pl.ds(s * INIT_SZ, INIT_SZ)],
                        agg_hbm.at[pl.ds(base_node + s * INIT_SZ, INIT_SZ)])

    @pl.when(s == NS - 1)
    def _():
        pltpu.sync_copy(acc.at[pl.ds((NS - 1) * INIT_SZ, INIT_LAST)],
                        agg_hbm.at[pl.ds(base_node + (NS - 1) * INIT_SZ, INIT_LAST)])


_sc_aggregate = functools.partial(
    pl.kernel,
    out_type=jax.ShapeDtypeStruct((N, C), jnp.float32),
    mesh=plsc.VectorSubcoreMesh(core_axis_name="c", subcore_axis_name="s"),
    compiler_params=pltpu.CompilerParams(use_tc_tiling_on_sc=False,
                                         needs_layout_passes=False),
    scratch_types=[
        pltpu.VMEM_SHARED((ACC_ROWS, C), jnp.float32),
        [pltpu.VMEM((CHUNK, C), jnp.float32)] * CPB,
        pltpu.VMEM((2, IDXBLK), jnp.int32),
        pltpu.VMEM((2, IDXBLK), jnp.int32),
        pltpu.VMEM((CCAP,), jnp.int32),
        pltpu.VMEM((CPB, CHUNK), jnp.int32),
        pltpu.VMEM((CPB, CHUNK), jnp.int32),
        pltpu.SemaphoreType.DMA,
        [pltpu.SemaphoreType.DMA] * CPB,
        [pltpu.SemaphoreType.DMA] * CPB,
    ],
)(_sc_body)


def _tc_body(norm_ref, agg_ref, w_ref, out_ref):
    h = norm_ref[...] * agg_ref[...]
    out_ref[...] = jnp.dot(h, w_ref[...], preferred_element_type=jnp.float32)


def _tc_matmul(norm, agg, W):
    return pl.pallas_call(
        _tc_body,
        grid=(N // ROWBLK,),
        in_specs=[
            pl.BlockSpec((ROWBLK, 1), lambda i: (i, 0)),
            pl.BlockSpec((ROWBLK, C), lambda i: (i, 0)),
            pl.BlockSpec((C, C), lambda i: (0, 0)),
        ],
        out_specs=pl.BlockSpec((ROWBLK, C), lambda i: (i, 0)),
        out_shape=jax.ShapeDtypeStruct((N, C), jnp.float32),
    )(norm, agg, W)


def kernel(x, sources, targets, norm, W):
    src = sources.astype(jnp.int32)
    tgt = targets.astype(jnp.int32)
    agg = _sc_aggregate(x, src, tgt)
    return _tc_matmul(norm, agg, W)


# speedup vs baseline: 1.0314x; 1.0001x over previous
import functools

import jax
import jax.numpy as jnp
from jax import lax
from jax.experimental import pallas as pl
from jax.experimental.pallas import tpu as pltpu
from jax.experimental.pallas import tpu_sc as plsc

N = 50000
E = 800000
C = 64

NC = 2
NS = 16
HALF = N // NC
ACC_ROWS = HALF + NS

EPS = E // NS
CHUNK = 96
CPB = 4
IDXBLK = CPB * CHUNK
NBLK = EPS // IDXBLK
TAIL = EPS - NBLK * IDXBLK
CCAP = 496
DUMP = 480

INIT_SZ = 1568
INIT_LAST = HALF - (NS - 1) * INIT_SZ

ROWBLK = 5000


def _sc_body(x_hbm, src_hbm, tgt_hbm, agg_hbm,
             acc, rowbufs, sblk, tblk, cpk, s2d, t2d,
             isem, gsems, ssems):
    c = lax.axis_index("c")
    s = lax.axis_index("s")
    base_node = c * HALF
    dummy = HALF + s

    @pl.when(s < NS - 1)
    def _():
        pltpu.sync_copy(x_hbm.at[pl.ds(base_node + s * INIT_SZ, INIT_SZ)],
                        acc.at[pl.ds(s * INIT_SZ, INIT_SZ)])

    @pl.when(s == NS - 1)
    def _():
        pltpu.sync_copy(x_hbm.at[pl.ds(base_node + (NS - 1) * INIT_SZ, INIT_LAST)],
                        acc.at[pl.ds((NS - 1) * INIT_SZ, INIT_LAST)])

    plsc.subcore_barrier()

    e0 = s * EPS

    def load_idx(p, blk, n):
        eb = e0 + blk * IDXBLK
        pltpu.async_copy(src_hbm.at[pl.ds(eb, n)], sblk.at[p].at[pl.ds(0, n)], isem)
        pltpu.async_copy(tgt_hbm.at[pl.ds(eb, n)], tblk.at[p].at[pl.ds(0, n)], isem)

    def drain_idx(p, n):
        pltpu.make_async_copy(src_hbm.at[pl.ds(e0, n)],
                              sblk.at[p].at[pl.ds(0, n)], isem).wait()
        pltpu.make_async_copy(tgt_hbm.at[pl.ds(e0, n)],
                              tblk.at[p].at[pl.ds(0, n)], isem).wait()

    iota16 = lax.iota(jnp.int32, 16)

    def compact(p, mvec_in, ngroups, g0=0):
        mvec = mvec_in
        for i in range(g0, g0 + ngroups):
            sv = sblk[p, pl.ds(i * 16, 16)]
            t = tblk[p, pl.ds(i * 16, 16)]
            tl = t - base_node
            ok = (tl >= 0) & (tl < HALF)
            key = jnp.where(ok, iota16, iota16 + 16)
            _, pk_c = plsc.sort_key_val(key, (sv << 15) | (tl & 32767))
            plsc.store_scatter(cpk, (mvec + iota16,), pk_c)
            mvec = mvec + plsc.all_reduce_population_count(ok)
        return mvec

    def fire_gather(k):
        return pltpu.async_copy(
            x_hbm.at[s2d.at[k]], rowbufs[k], gsems[k])

    def wait_gather(k):
        pltpu.make_async_copy(
            x_hbm.at[s2d.at[k]], rowbufs[k], gsems[k]).wait()

    def fire_scatter(k):
        pltpu.async_copy(rowbufs[k], acc.at[t2d.at[k]], ssems[k], add=True)

    def drain_scatter(k):
        pltpu.make_async_copy(rowbufs[k], acc.at[t2d.at[k]], ssems[k]).wait()

    def fire_block(m_tot, pend):
        nfire = m_tot // CHUNK
        for k in range(CPB):
            @pl.when(k < nfire)
            def _():
                @pl.when(((pend >> k) & 1) == 1)
                def _():
                    drain_scatter(k)
                for ii in range(CHUNK // 16):
                    pk = cpk[pl.ds(k * CHUNK + ii * 16, 16)]
                    s2d[k, pl.ds(ii * 16, 16)] = pk >> 15
                    t2d[k, pl.ds(ii * 16, 16)] = pk & 32767
                fire_gather(k)

        mrem = m_tot - nfire * CHUNK

        @pl.when(nfire > 0)
        def _():
            for i in range(CHUNK // 16):
                @pl.when(i * 16 < mrem)
                def _():
                    src_pos = nfire * CHUNK + i * 16 + iota16
                    cpk[pl.ds(i * 16, 16)] = plsc.load_gather(cpk, (src_pos,))

        pend_out = pend & ~((jnp.int32(1) << nfire) - 1)
        return (nfire.astype(jnp.int32), mrem.astype(jnp.int32),
                pend_out.astype(jnp.int32))

    def scatter_block(nprev, pend):
        for k in range(CPB):
            @pl.when(k < nprev)
            def _():
                wait_gather(k)
                fire_scatter(k)
        return (pend | ((jnp.int32(1) << nprev) - 1)).astype(jnp.int32)

    pltpu.sync_copy(src_hbm.at[pl.ds(e0, IDXBLK)], sblk.at[0])
    pltpu.sync_copy(tgt_hbm.at[pl.ds(e0, IDXBLK)], tblk.at[0])
    load_idx(1, 1, IDXBLK)
    load_idx(2, 2, IDXBLK)
    mvec = compact(0, jnp.zeros((16,), jnp.int32), IDXBLK // 16)
    nfire, _, pend = fire_block(jnp.max(mvec), jnp.int32(0))
    mvec = mvec - nfire * CHUNK

    def body(g, carry):
        mvec, nprev, pend = carry
        p = g % 3
        drain_idx(p, IDXBLK)

        @pl.when(g + 2 < NBLK)
        def _():
            load_idx((g + 2) % 3, g + 2, IDXBLK)

        mvec = compact(p, mvec, IDXBLK // 32)
        pend = scatter_block(nprev, pend)
        mvec = compact(p, mvec, IDXBLK // 16 - IDXBLK // 32, g0=IDXBLK // 32)
        nfire, _, pend = fire_block(jnp.max(mvec), pend)
        return mvec - nfire * CHUNK, nfire, pend

    mvec, nprev, pend = lax.fori_loop(1, NBLK, body, (mvec, nfire, pend))

    pend = scatter_block(nprev, pend)

    et = e0 + NBLK * IDXBLK
    pltpu.sync_copy(src_hbm.at[pl.ds(et, TAIL)], sblk.at[0].at[pl.ds(0, TAIL)])
    pltpu.sync_copy(tgt_hbm.at[pl.ds(et, TAIL)], tblk.at[0].at[pl.ds(0, TAIL)])
    mvec = compact(0, mvec, TAIL // 16)
    m_tot = jnp.max(mvec)
    nfire2 = (m_tot + CHUNK - 1) // CHUNK
    for i in range((2 * CHUNK) // 16):
        lane = lax.iota(jnp.int32, 16) + (i * 16)
        inside = lane < m_tot
        cpk[pl.ds(i * 16, 16)] = jnp.where(inside, cpk[pl.ds(i * 16, 16)], dummy)
    for k in range(2):
        @pl.when(((pend >> k) & 1) == 1)
        def _():
            drain_scatter(k)
    pend = pend & ~3
    for k in range(2):
        @pl.when(k < nfire2)
        def _():
            for ii in range(CHUNK // 16):
                pk = cpk[pl.ds(k * CHUNK + ii * 16, 16)]
                s2d[k, pl.ds(ii * 16, 16)] = pk >> 15
                t2d[k, pl.ds(ii * 16, 16)] = pk & 32767
            fire_gather(k)
    for k in range(2):
        @pl.when(k < nfire2)
        def _():
            wait_gather(k)
            fire_scatter(k)
    pend = pend | ((jnp.int32(1) << nfire2) - 1)

    for k in range(CPB):
        @pl.when(((pend >> k) & 1) == 1)
        def _():
            drain_scatter(k)

    plsc.subcore_barrier()

    @pl.when(s < NS - 1)
    def _():
        pltpu.sync_copy(acc.at[pl.ds(s * INIT_SZ, INIT_SZ)],
                        agg_hbm.at[pl.ds(base_node + s * INIT_SZ, INIT_SZ)])

    @pl.when(s == NS - 1)
    def _():
        pltpu.sync_copy(acc.at[pl.ds((NS - 1) * INIT_SZ, INIT_LAST)],
                        agg_hbm.at[pl.ds(base_node + (NS - 1) * INIT_SZ, INIT_LAST)])


_sc_aggregate = functools.partial(
    pl.kernel,
    out_type=jax.ShapeDtypeStruct((N, C), jnp.float32),
    mesh=plsc.VectorSubcoreMesh(core_axis_name="c", subcore_axis_name="s"),
    compiler_params=pltpu.CompilerParams(use_tc_tiling_on_sc=False,
                                         needs_layout_passes=False),
    scratch_types=[
        pltpu.VMEM_SHARED((ACC_ROWS, C), jnp.float32),
        [pltpu.VMEM((CHUNK, C), jnp.float32)] * CPB,
        pltpu.VMEM((3, IDXBLK), jnp.int32),
        pltpu.VMEM((3, IDXBLK), jnp.int32),
        pltpu.VMEM((CCAP,), jnp.int32),
        pltpu.VMEM((CPB, CHUNK), jnp.int32),
        pltpu.VMEM((CPB, CHUNK), jnp.int32),
        pltpu.SemaphoreType.DMA,
        [pltpu.SemaphoreType.DMA] * CPB,
        [pltpu.SemaphoreType.DMA] * CPB,
    ],
)(_sc_body)


def _tc_body(norm_ref, agg_ref, w_ref, out_ref):
    h = norm_ref[...] * agg_ref[...]
    out_ref[...] = jnp.dot(h, w_ref[...], preferred_element_type=jnp.float32)


def _tc_matmul(norm, agg, W):
    return pl.pallas_call(
        _tc_body,
        grid=(N // ROWBLK,),
        in_specs=[
            pl.BlockSpec((ROWBLK, 1), lambda i: (i, 0)),
            pl.BlockSpec((ROWBLK, C), lambda i: (i, 0)),
            pl.BlockSpec((C, C), lambda i: (0, 0)),
        ],
        out_specs=pl.BlockSpec((ROWBLK, C), lambda i: (i, 0)),
        out_shape=jax.ShapeDtypeStruct((N, C), jnp.float32),
    )(norm, agg, W)


def kernel(x, sources, targets, norm, W):
    src = sources.astype(jnp.int32)
    tgt = targets.astype(jnp.int32)
    agg = _sc_aggregate(x, src, tgt)
    return _tc_matmul(norm, agg, W)


# speedup vs baseline: 1.1099x; 1.0761x over previous
import functools

import jax
import jax.numpy as jnp
from jax import lax
from jax.experimental import pallas as pl
from jax.experimental.pallas import tpu as pltpu
from jax.experimental.pallas import tpu_sc as plsc

N = 50000
E = 800000
C = 64

NC = 2
NS = 16
HALF = N // NC
ACC_ROWS = HALF + NS

EPS = E // NS
CHUNK = 96
CPB = 4
IDXBLK = 480
NBLK = EPS // IDXBLK
TAIL = EPS - NBLK * IDXBLK
CCAP = 592

INIT_SZ = 1568
INIT_LAST = HALF - (NS - 1) * INIT_SZ

ROWBLK = 5000


def _sc_body(x_hbm, src_hbm, tgt_hbm, agg_hbm,
             acc, rowbufs, sblk, tblk, cpk, s2d, t2d,
             isem, gsems, ssems):
    c = lax.axis_index("c")
    s = lax.axis_index("s")
    base_node = c * HALF
    dummy = HALF + s

    @pl.when(s < NS - 1)
    def _():
        pltpu.sync_copy(x_hbm.at[pl.ds(base_node + s * INIT_SZ, INIT_SZ)],
                        acc.at[pl.ds(s * INIT_SZ, INIT_SZ)])

    @pl.when(s == NS - 1)
    def _():
        pltpu.sync_copy(x_hbm.at[pl.ds(base_node + (NS - 1) * INIT_SZ, INIT_LAST)],
                        acc.at[pl.ds((NS - 1) * INIT_SZ, INIT_LAST)])

    plsc.subcore_barrier()

    e0 = s * EPS

    def load_idx(p, blk, n):
        eb = e0 + blk * IDXBLK
        pltpu.async_copy(src_hbm.at[pl.ds(eb, n)], sblk.at[p].at[pl.ds(0, n)], isem)
        pltpu.async_copy(tgt_hbm.at[pl.ds(eb, n)], tblk.at[p].at[pl.ds(0, n)], isem)

    def drain_idx(p, n):
        pltpu.make_async_copy(src_hbm.at[pl.ds(e0, n)],
                              sblk.at[p].at[pl.ds(0, n)], isem).wait()
        pltpu.make_async_copy(tgt_hbm.at[pl.ds(e0, n)],
                              tblk.at[p].at[pl.ds(0, n)], isem).wait()

    iota16 = lax.iota(jnp.int32, 16)

    def compact(p, mvec_in, ngroups, g0=0):
        mvec = mvec_in
        for i in range(g0, g0 + ngroups):
            sv = sblk[p, pl.ds(i * 16, 16)]
            t = tblk[p, pl.ds(i * 16, 16)]
            tl = t - base_node
            ok = (tl >= 0) & (tl < HALF)
            key = jnp.where(ok, iota16, iota16 + 16)
            _, pk_c = plsc.sort_key_val(key, (sv << 15) | (tl & 32767))
            plsc.store_scatter(cpk, (mvec + iota16,), pk_c)
            mvec = mvec + plsc.all_reduce_population_count(ok)
        return mvec

    def fire_gather(k):
        return pltpu.async_copy(
            x_hbm.at[s2d.at[k]], rowbufs[k], gsems[k])

    def wait_gather(k):
        pltpu.make_async_copy(
            x_hbm.at[s2d.at[k]], rowbufs[k], gsems[k]).wait()

    def fire_scatter(k):
        pltpu.async_copy(rowbufs[k], acc.at[t2d.at[k]], ssems[k], add=True)

    def drain_scatter(k):
        pltpu.make_async_copy(rowbufs[k], acc.at[t2d.at[k]], ssems[k]).wait()

    def fire_block(m_tot, pend):
        nfire = m_tot // CHUNK
        @pl.when(nfire > CPB)
        def _():
            @pl.when((pend & 1) == 1)
            def _():
                drain_scatter(0)
            for ii in range(CHUNK // 16):
                pk = cpk[pl.ds(CPB * CHUNK + ii * 16, 16)]
                s2d[0, pl.ds(ii * 16, 16)] = pk >> 15
                t2d[0, pl.ds(ii * 16, 16)] = pk & 32767
            pltpu.async_copy(x_hbm.at[s2d.at[0]], rowbufs[0], gsems[0]).wait()
            pltpu.sync_copy(rowbufs[0], acc.at[t2d.at[0]], add=True)

        pend = jnp.where(nfire > CPB, pend & ~1, pend)
        for k in range(CPB):
            @pl.when(k < nfire)
            def _():
                @pl.when(((pend >> k) & 1) == 1)
                def _():
                    drain_scatter(k)
                for ii in range(CHUNK // 16):
                    pk = cpk[pl.ds(k * CHUNK + ii * 16, 16)]
                    s2d[k, pl.ds(ii * 16, 16)] = pk >> 15
                    t2d[k, pl.ds(ii * 16, 16)] = pk & 32767
                fire_gather(k)

        mrem = m_tot - nfire * CHUNK

        @pl.when(nfire > 0)
        def _():
            for i in range(CHUNK // 16):
                @pl.when(i * 16 < mrem)
                def _():
                    src_pos = nfire * CHUNK + i * 16 + iota16
                    cpk[pl.ds(i * 16, 16)] = plsc.load_gather(cpk, (src_pos,))

        pend_out = pend & ~((jnp.int32(1) << nfire) - 1)
        return (nfire.astype(jnp.int32), mrem.astype(jnp.int32),
                pend_out.astype(jnp.int32))

    def scatter_block(nprev, pend):
        for k in range(CPB):
            @pl.when(k < nprev)
            def _():
                wait_gather(k)
                fire_scatter(k)
        return (pend | ((jnp.int32(1) << nprev) - 1)).astype(jnp.int32)

    pltpu.sync_copy(src_hbm.at[pl.ds(e0, IDXBLK)], sblk.at[0])
    pltpu.sync_copy(tgt_hbm.at[pl.ds(e0, IDXBLK)], tblk.at[0])
    load_idx(1, 1, IDXBLK)
    load_idx(2, 2, IDXBLK)
    mvec = compact(0, jnp.zeros((16,), jnp.int32), IDXBLK // 16)
    nfire, _, pend = fire_block(jnp.max(mvec), jnp.int32(0))
    mvec = mvec - nfire * CHUNK

    def body(g, carry):
        mvec, nprev, pend = carry
        p = g % 3
        drain_idx(p, IDXBLK)

        @pl.when(g + 2 < NBLK)
        def _():
            load_idx((g + 2) % 3, g + 2, IDXBLK)

        mvec = compact(p, mvec, 15)
        pend = scatter_block(nprev, pend)
        mvec = compact(p, mvec, 15, g0=15)
        nfire, _, pend = fire_block(jnp.max(mvec), pend)
        return mvec - nfire * CHUNK, nfire, pend

    mvec, nprev, pend = lax.fori_loop(1, NBLK, body, (mvec, nfire, pend))

    pend = scatter_block(nprev, pend)

    et = e0 + NBLK * IDXBLK
    pltpu.sync_copy(src_hbm.at[pl.ds(et, TAIL)], sblk.at[0].at[pl.ds(0, TAIL)])
    pltpu.sync_copy(tgt_hbm.at[pl.ds(et, TAIL)], tblk.at[0].at[pl.ds(0, TAIL)])
    mvec = compact(0, mvec, TAIL // 16)
    m_tot = jnp.max(mvec)
    nfire2 = (m_tot + CHUNK - 1) // CHUNK
    for i in range((2 * CHUNK) // 16):
        lane = lax.iota(jnp.int32, 16) + (i * 16)
        inside = lane < m_tot
        cpk[pl.ds(i * 16, 16)] = jnp.where(inside, cpk[pl.ds(i * 16, 16)], dummy)
    for k in range(2):
        @pl.when(((pend >> k) & 1) == 1)
        def _():
            drain_scatter(k)
    pend = pend & ~3
    for k in range(2):
        @pl.when(k < nfire2)
        def _():
            for ii in range(CHUNK // 16):
                pk = cpk[pl.ds(k * CHUNK + ii * 16, 16)]
                s2d[k, pl.ds(ii * 16, 16)] = pk >> 15
                t2d[k, pl.ds(ii * 16, 16)] = pk & 32767
            fire_gather(k)
    for k in range(2):
        @pl.when(k < nfire2)
        def _():
            wait_gather(k)
            fire_scatter(k)
    pend = pend | ((jnp.int32(1) << nfire2) - 1)

    for k in range(CPB):
        @pl.when(((pend >> k) & 1) == 1)
        def _():
            drain_scatter(k)

    plsc.subcore_barrier()

    @pl.when(s < NS - 1)
    def _():
        pltpu.sync_copy(acc.at[pl.ds(s * INIT_SZ, INIT_SZ)],
                        agg_hbm.at[pl.ds(base_node + s * INIT_SZ, INIT_SZ)])

    @pl.when(s == NS - 1)
    def _():
        pltpu.sync_copy(acc.at[pl.ds((NS - 1) * INIT_SZ, INIT_LAST)],
                        agg_hbm.at[pl.ds(base_node + (NS - 1) * INIT_SZ, INIT_LAST)])


_sc_aggregate = functools.partial(
    pl.kernel,
    out_type=jax.ShapeDtypeStruct((N, C), jnp.float32),
    mesh=plsc.VectorSubcoreMesh(core_axis_name="c", subcore_axis_name="s"),
    compiler_params=pltpu.CompilerParams(use_tc_tiling_on_sc=False,
                                         needs_layout_passes=False),
    scratch_types=[
        pltpu.VMEM_SHARED((ACC_ROWS, C), jnp.float32),
        [pltpu.VMEM((CHUNK, C), jnp.float32)] * CPB,
        pltpu.VMEM((3, IDXBLK), jnp.int32),
        pltpu.VMEM((3, IDXBLK), jnp.int32),
        pltpu.VMEM((CCAP,), jnp.int32),
        pltpu.VMEM((CPB, CHUNK), jnp.int32),
        pltpu.VMEM((CPB, CHUNK), jnp.int32),
        pltpu.SemaphoreType.DMA,
        [pltpu.SemaphoreType.DMA] * CPB,
        [pltpu.SemaphoreType.DMA] * CPB,
    ],
)(_sc_body)


def _tc_body(norm_ref, agg_ref, w_ref, out_ref):
    h = norm_ref[...] * agg_ref[...]
    out_ref[...] = jnp.dot(h, w_ref[...], preferred_element_type=jnp.float32)


def _tc_matmul(norm, agg, W):
    return pl.pallas_call(
        _tc_body,
        grid=(N // ROWBLK,),
        in_specs=[
            pl.BlockSpec((ROWBLK, 1), lambda i: (i, 0)),
            pl.BlockSpec((ROWBLK, C), lambda i: (i, 0)),
            pl.BlockSpec((C, C), lambda i: (0, 0)),
        ],
        out_specs=pl.BlockSpec((ROWBLK, C), lambda i: (i, 0)),
        out_shape=jax.ShapeDtypeStruct((N, C), jnp.float32),
    )(norm, agg, W)


def kernel(x, sources, targets, norm, W):
    src = sources.astype(jnp.int32)
    tgt = targets.astype(jnp.int32)
    agg = _sc_aggregate(x, src, tgt)
    return _tc_matmul(norm, agg, W)


# speedup vs baseline: 1.1817x; 1.0647x over previous
import functools

import jax
import jax.numpy as jnp
from jax import lax
from jax.experimental import pallas as pl
from jax.experimental.pallas import tpu as pltpu
from jax.experimental.pallas import tpu_sc as plsc

N = 50000
E = 800000
C = 64

NC = 2
NS = 16
HALF = N // NC
ACC_ROWS = HALF + NS

EPS = E // NS
CHUNK = 96
CPB = 4
IDXBLK = 624
NBLK = EPS // IDXBLK
TAIL = EPS - NBLK * IDXBLK
CCAP = 736
SLOWMAX = (IDXBLK + CHUNK - 1) // CHUNK

INIT_SZ = 1568
INIT_LAST = HALF - (NS - 1) * INIT_SZ

ROWBLK = 5000


def _sc_body(x_hbm, src_hbm, tgt_hbm, agg_hbm,
             acc, rowbufs, sblk, tblk, cpk, s2d, t2d,
             isem, gsems, ssems):
    c = lax.axis_index("c")
    s = lax.axis_index("s")
    base_node = c * HALF
    dummy = HALF + s

    @pl.when(s < NS - 1)
    def _():
        pltpu.sync_copy(x_hbm.at[pl.ds(base_node + s * INIT_SZ, INIT_SZ)],
                        acc.at[pl.ds(s * INIT_SZ, INIT_SZ)])

    @pl.when(s == NS - 1)
    def _():
        pltpu.sync_copy(x_hbm.at[pl.ds(base_node + (NS - 1) * INIT_SZ, INIT_LAST)],
                        acc.at[pl.ds((NS - 1) * INIT_SZ, INIT_LAST)])

    plsc.subcore_barrier()

    e0 = s * EPS

    def load_idx(p, blk, n):
        eb = e0 + blk * IDXBLK
        pltpu.async_copy(src_hbm.at[pl.ds(eb, n)], sblk.at[p].at[pl.ds(0, n)], isem)
        pltpu.async_copy(tgt_hbm.at[pl.ds(eb, n)], tblk.at[p].at[pl.ds(0, n)], isem)

    def drain_idx(p, n):
        pltpu.make_async_copy(src_hbm.at[pl.ds(e0, n)],
                              sblk.at[p].at[pl.ds(0, n)], isem).wait()
        pltpu.make_async_copy(tgt_hbm.at[pl.ds(e0, n)],
                              tblk.at[p].at[pl.ds(0, n)], isem).wait()

    iota16 = lax.iota(jnp.int32, 16)

    def compact(p, mvec_in, ngroups, g0=0):
        mvec = mvec_in
        for i in range(g0, g0 + ngroups):
            sv = sblk[p, pl.ds(i * 16, 16)]
            t = tblk[p, pl.ds(i * 16, 16)]
            tl = t - base_node
            ok = (tl >= 0) & (tl < HALF)
            key = jnp.where(ok, iota16, iota16 + 16)
            _, pk_c = plsc.sort_key_val(key, (sv << 15) | (tl & 32767))
            plsc.store_scatter(cpk, (mvec + iota16,), pk_c)
            mvec = mvec + plsc.all_reduce_population_count(ok)
        return mvec

    def fire_gather(k):
        return pltpu.async_copy(
            x_hbm.at[s2d.at[k]], rowbufs[k], gsems[k])

    def wait_gather(k):
        pltpu.make_async_copy(
            x_hbm.at[s2d.at[k]], rowbufs[k], gsems[k]).wait()

    def fire_scatter(k):
        pltpu.async_copy(rowbufs[k], acc.at[t2d.at[k]], ssems[k], add=True)

    def drain_scatter(k):
        pltpu.make_async_copy(rowbufs[k], acc.at[t2d.at[k]], ssems[k]).wait()

    def fire_block(m_tot, pend):
        nfire = m_tot // CHUNK
        @pl.when(nfire > CPB)
        def _():
            @pl.when((pend & 1) == 1)
            def _():
                drain_scatter(0)

        for kslow in range(CPB, SLOWMAX):
            @pl.when(kslow < nfire)
            def _():
                for ii in range(CHUNK // 16):
                    pk = cpk[pl.ds(kslow * CHUNK + ii * 16, 16)]
                    s2d[0, pl.ds(ii * 16, 16)] = pk >> 15
                    t2d[0, pl.ds(ii * 16, 16)] = pk & 32767
                pltpu.async_copy(x_hbm.at[s2d.at[0]], rowbufs[0], gsems[0]).wait()
                pltpu.sync_copy(rowbufs[0], acc.at[t2d.at[0]], add=True)

        pend = jnp.where(nfire > CPB, pend & ~1, pend)
        for k in range(CPB):
            @pl.when(k < nfire)
            def _():
                @pl.when(((pend >> k) & 1) == 1)
                def _():
                    drain_scatter(k)
                for ii in range(CHUNK // 16):
                    pk = cpk[pl.ds(k * CHUNK + ii * 16, 16)]
                    s2d[k, pl.ds(ii * 16, 16)] = pk >> 15
                    t2d[k, pl.ds(ii * 16, 16)] = pk & 32767
                fire_gather(k)

        mrem = m_tot - nfire * CHUNK

        @pl.when(nfire > 0)
        def _():
            for i in range(CHUNK // 16):
                @pl.when(i * 16 < mrem)
                def _():
                    src_pos = nfire * CHUNK + i * 16 + iota16
                    cpk[pl.ds(i * 16, 16)] = plsc.load_gather(cpk, (src_pos,))

        pend_out = pend & ~((jnp.int32(1) << nfire) - 1)
        return (nfire.astype(jnp.int32), mrem.astype(jnp.int32),
                pend_out.astype(jnp.int32))

    def scatter_block(nprev, pend):
        for k in range(CPB):
            @pl.when(k < nprev)
            def _():
                wait_gather(k)
                fire_scatter(k)
        return (pend | ((jnp.int32(1) << nprev) - 1)).astype(jnp.int32)

    pltpu.sync_copy(src_hbm.at[pl.ds(e0, IDXBLK)], sblk.at[0])
    pltpu.sync_copy(tgt_hbm.at[pl.ds(e0, IDXBLK)], tblk.at[0])
    load_idx(1, 1, IDXBLK)
    load_idx(2, 2, IDXBLK)
    mvec = compact(0, jnp.zeros((16,), jnp.int32), IDXBLK // 16)
    nfire, _, pend = fire_block(jnp.max(mvec), jnp.int32(0))
    mvec = mvec - nfire * CHUNK

    def body(g, carry):
        mvec, nprev, pend = carry
        p = g % 3
        drain_idx(p, IDXBLK)

        @pl.when(g + 2 < NBLK)
        def _():
            load_idx((g + 2) % 3, g + 2, IDXBLK)

        mvec = compact(p, mvec, 19)
        pend = scatter_block(nprev, pend)
        mvec = compact(p, mvec, 20, g0=19)
        nfire, _, pend = fire_block(jnp.max(mvec), pend)
        return mvec - nfire * CHUNK, nfire, pend

    mvec, nprev, pend = lax.fori_loop(1, NBLK, body, (mvec, nfire, pend))

    pend = scatter_block(nprev, pend)

    et = e0 + NBLK * IDXBLK
    pltpu.sync_copy(src_hbm.at[pl.ds(et, TAIL)], sblk.at[0].at[pl.ds(0, TAIL)])
    pltpu.sync_copy(tgt_hbm.at[pl.ds(et, TAIL)], tblk.at[0].at[pl.ds(0, TAIL)])
    mvec = compact(0, mvec, TAIL // 16)
    m_tot = jnp.max(mvec)
    nfire2 = (m_tot + CHUNK - 1) // CHUNK
    for i in range((2 * CHUNK) // 16):
        lane = lax.iota(jnp.int32, 16) + (i * 16)
        inside = lane < m_tot
        cpk[pl.ds(i * 16, 16)] = jnp.where(inside, cpk[pl.ds(i * 16, 16)], dummy)
    for k in range(2):
        @pl.when(((pend >> k) & 1) == 1)
        def _():
            drain_scatter(k)
    pend = pend & ~3
    for k in range(2):
        @pl.when(k < nfire2)
        def _():
            for ii in range(CHUNK // 16):
                pk = cpk[pl.ds(k * CHUNK + ii * 16, 16)]
                s2d[k, pl.ds(ii * 16, 16)] = pk >> 15
                t2d[k, pl.ds(ii * 16, 16)] = pk & 32767
            fire_gather(k)
    for k in range(2):
        @pl.when(k < nfire2)
        def _():
            wait_gather(k)
            fire_scatter(k)
    pend = pend | ((jnp.int32(1) << nfire2) - 1)

    for k in range(CPB):
        @pl.when(((pend >> k) & 1) == 1)
        def _():
            drain_scatter(k)

    plsc.subcore_barrier()

    @pl.when(s < NS - 1)
    def _():
        pltpu.sync_copy(acc.at[pl.ds(s * INIT_SZ, INIT_SZ)],
                        agg_hbm.at[pl.ds(base_node + s * INIT_SZ, INIT_SZ)])

    @pl.when(s == NS - 1)
    def _():
        pltpu.sync_copy(acc.at[pl.ds((NS - 1) * INIT_SZ, INIT_LAST)],
                        agg_hbm.at[pl.ds(base_node + (NS - 1) * INIT_SZ, INIT_LAST)])


_sc_aggregate = functools.partial(
    pl.kernel,
    out_type=jax.ShapeDtypeStruct((N, C), jnp.float32),
    mesh=plsc.VectorSubcoreMesh(core_axis_name="c", subcore_axis_name="s"),
    compiler_params=pltpu.CompilerParams(use_tc_tiling_on_sc=False,
                                         needs_layout_passes=False),
    scratch_types=[
        pltpu.VMEM_SHARED((ACC_ROWS, C), jnp.float32),
        [pltpu.VMEM((CHUNK, C), jnp.float32)] * CPB,
        pltpu.VMEM((3, IDXBLK), jnp.int32),
        pltpu.VMEM((3, IDXBLK), jnp.int32),
        pltpu.VMEM((CCAP,), jnp.int32),
        pltpu.VMEM((CPB, CHUNK), jnp.int32),
        pltpu.VMEM((CPB, CHUNK), jnp.int32),
        pltpu.SemaphoreType.DMA,
        [pltpu.SemaphoreType.DMA] * CPB,
        [pltpu.SemaphoreType.DMA] * CPB,
    ],
)(_sc_body)


def _tc_body(norm_ref, agg_ref, w_ref, out_ref):
    h = norm_ref[...] * agg_ref[...]
    out_ref[...] = jnp.dot(h, w_ref[...], preferred_element_type=jnp.float32)


def _tc_matmul(norm, agg, W):
    return pl.pallas_call(
        _tc_body,
        grid=(N // ROWBLK,),
        in_specs=[
            pl.BlockSpec((ROWBLK, 1), lambda i: (i, 0)),
            pl.BlockSpec((ROWBLK, C), lambda i: (i, 0)),
            pl.BlockSpec((C, C), lambda i: (0, 0)),
        ],
        out_specs=pl.BlockSpec((ROWBLK, C), lambda i: (i, 0)),
        out_shape=jax.ShapeDtypeStruct((N, C), jnp.float32),
    )(norm, agg, W)


def kernel(x, sources, targets, norm, W):
    src = sources.astype(jnp.int32)
    tgt = targets.astype(jnp.int32)
    agg = _sc_aggregate(x, src, tgt)
    return _tc_matmul(norm, agg, W)
